# baseline (speedup 1.0000x reference)
import functools

import jax
import jax.numpy as jnp
from jax import lax
from jax.experimental import pallas as pl
from jax.experimental.pallas import tpu as pltpu
from jax.experimental.pallas import tpu_sc as plsc

C = 32
CW = C // 2
D = 128
H = 128
W = 128
DHW = D * H * W

P = 128
L = 16
NW = 32

MASK_HI = -65536


def _sc_grid_sample(xT, table, n):
    num_chunks = n // P
    iters = (num_chunks + NW - 1) // NW

    mesh = plsc.VectorSubcoreMesh(core_axis_name="c", subcore_axis_name="s")

    scratch = (
        [pltpu.VMEM((3, P), jnp.float32) for _ in range(2)]
        + [pltpu.VMEM((3, P), jnp.float32) for _ in range(2)]
        + [pltpu.VMEM((P, C), jnp.float32) for _ in range(2)]
        + [pltpu.VMEM((P,), jnp.int32) for _ in range(16)]
        + [pltpu.VMEM((P, CW), jnp.int32) for _ in range(16)]
        + [pltpu.SemaphoreType.DMA for _ in range(6)]
    )

    @functools.partial(
        pl.kernel,
        out_type=jax.ShapeDtypeStruct((n, C), jnp.float32),
        mesh=mesh,
        compiler_params=pltpu.CompilerParams(use_tc_tiling_on_sc=False),
        scratch_types=scratch,
    )
    def k(xT_hbm, tab_hbm, out_hbm, *s):
        crd = s[0:2]
        wbb = s[2:4]
        outt = s[4:6]
        idx = [s[6:14], s[14:22]]
        rows = [s[22:30], s[30:38]]
        sc_sem = s[38:40]
        sg_sem = s[40:42]
        so_sem = s[42:44]
        wid = lax.axis_index("s") * 2 + lax.axis_index("c")

        def fire_coords(t, b):
            cid = jnp.minimum(wid + t * NW, num_chunks - 1)
            base = pl.multiple_of(cid * P, P)
            pltpu.async_copy(xT_hbm.at[:, pl.ds(base, P)], crd[b], sc_sem[b])

        def jw_maker(b):
            cb, wb = crd[b], wbb[b]
            ib = idx[b]

            def jw_body(j, c2):
                sj = pl.ds(j * L, L)
                fx = (cb[0, sj] + 1.0) * (0.5 * (W - 1))
                fy = (cb[1, sj] + 1.0) * (0.5 * (H - 1))
                fz = (cb[2, sj] + 1.0) * (0.5 * (D - 1))
                fx = jnp.minimum(jnp.maximum(fx, 0.0), float(W - 1))
                fy = jnp.minimum(jnp.maximum(fy, 0.0), float(H - 1))
                fz = jnp.minimum(jnp.maximum(fz, 0.0), float(D - 1))
                x0 = fx.astype(jnp.int32)
                y0 = fy.astype(jnp.int32)
                z0 = fz.astype(jnp.int32)
                wb[0, sj] = fx - x0.astype(jnp.float32)
                wb[1, sj] = fy - y0.astype(jnp.float32)
                wb[2, sj] = fz - z0.astype(jnp.float32)
                x1 = jnp.minimum(x0 + 1, W - 1)
                y1 = jnp.minimum(y0 + 1, H - 1)
                z1 = jnp.minimum(z0 + 1, D - 1)
                zl = z0 * (H * W)
                zh = z1 * (H * W)
                yl = y0 * W
                yh = y1 * W
                ib[0][sj] = zl + yl + x0
                ib[1][sj] = zl + yl + x1
                ib[2][sj] = zl + yh + x0
                ib[3][sj] = zl + yh + x1
                ib[4][sj] = zh + yl + x0
                ib[5][sj] = zh + yl + x1
                ib[6][sj] = zh + yh + x0
                ib[7][sj] = zh + yh + x1
                return c2

            return jw_body

        def unpack2(w16):
            lo = lax.bitcast_convert_type(
                lax.shift_left(w16, 16), jnp.float32)
            hi = lax.bitcast_convert_type(
                lax.bitwise_and(w16, jnp.int32(MASK_HI)), jnp.float32)
            return lo, hi

        def comb_maker(b):
            wb = wbb[b]
            rb = rows[b]
            ob = outt[b]

            def comb_body(j, c2):
                sj = pl.ds(j * L, L)
                wxv = wb[0, sj]
                wyv = wb[1, sj]
                wzv = wb[2, sj]
                for q in range(L):
                    p = j * L + q
                    bwx = jnp.full((L,), wxv[q])
                    bwy = jnp.full((L,), wyv[q])
                    bwz = jnp.full((L,), wzv[q])
                    cs = [unpack2(rb[kc][p, :]) for kc in range(8)]
                    for h in range(2):
                        c000 = cs[0][h]
                        c001 = cs[1][h]
                        c010 = cs[2][h]
                        c011 = cs[3][h]
                        c100 = cs[4][h]
                        c101 = cs[5][h]
                        c110 = cs[6][h]
                        c111 = cs[7][h]
                        c00 = c000 + bwx * (c001 - c000)
                        c01 = c010 + bwx * (c011 - c010)
                        c10 = c100 + bwx * (c101 - c100)
                        c11 = c110 + bwx * (c111 - c110)
                        c0 = c00 + bwy * (c01 - c00)
                        c1 = c10 + bwy * (c11 - c10)
                        ob[p, pl.ds(h * L, L)] = c0 + bwz * (c1 - c0)
                return c2

            return comb_body

        def stage_a(t, b):
            cid = wid + t * NW

            @pl.when(cid < num_chunks)
            def _():
                pltpu.make_async_copy(
                    xT_hbm.at[:, pl.ds(0, P)], crd[b], sc_sem[b]).wait()
                lax.fori_loop(0, P // L, jw_maker(b), 0)
                for kc in range(8):
                    pltpu.async_copy(
                        tab_hbm.at[idx[b][kc]], rows[b][kc], sg_sem[b])
                fire_coords(t + 2, b)

        def stage_b(t, b):
            cid = wid + t * NW

            @pl.when(cid < num_chunks)
            def _():
                base = pl.multiple_of(cid * P, P)
                for kc in range(8):
                    pltpu.make_async_copy(
                        tab_hbm.at[idx[b][kc]], rows[b][kc], sg_sem[b]).wait()

                @pl.when(t >= 2)
                def _w():
                    pltpu.make_async_copy(
                        out_hbm.at[pl.ds(0, P), :], outt[b], so_sem[b]).wait()

                lax.fori_loop(0, P // L, comb_maker(b), 0)
                pltpu.async_copy(
                    outt[b], out_hbm.at[pl.ds(base, P), :], so_sem[b])

        fire_coords(0, 0)
        fire_coords(1, 1)
        stage_a(0, 0)

        def u_body(u, carry):
            t0 = u * 2
            stage_a(t0 + 1, 1)
            stage_b(t0, 0)
            stage_a(t0 + 2, 0)
            stage_b(t0 + 1, 1)
            return carry

        lax.fori_loop(0, (iters + 1) // 2, u_body, 0)

        for b in (0, 1):
            pltpu.make_async_copy(
                out_hbm.at[pl.ds(0, P), :], outt[b], so_sem[b]).wait()
            pltpu.make_async_copy(
                xT_hbm.at[:, pl.ds(0, P)], crd[b], sc_sem[b]).wait()

    return k(xT, table)


def kernel(x, feature_grid):
    n = x.shape[0]
    n_pad = ((n + P - 1) // P) * P
    tb = feature_grid[0].astype(jnp.bfloat16).reshape(C, DHW).T
    table = lax.bitcast_convert_type(
        tb.reshape(DHW, CW, 2), jnp.int32)
    xT = x.T
    if n_pad != n:
        xT = jnp.pad(xT, ((0, 0), (0, n_pad - n)))
    out = _sc_grid_sample(xT, table, n_pad)
    out = out.reshape(n_pad, 2, CW).swapaxes(1, 2).reshape(n_pad, C)
    return out[:n].T.reshape(1, C, 1, 1, n)


# speedup vs baseline: 5.3218x; 5.3218x over previous
import functools

import jax
import jax.numpy as jnp
from jax import lax
from jax.experimental import pallas as pl
from jax.experimental.pallas import tpu as pltpu
from jax.experimental.pallas import tpu_sc as plsc

C = 32
D = 128
H = 128
W = 128
DHW = D * H * W

P = 128
L = 16
NW = 32


def _sc_grid_sample(xT, table, n):
    num_chunks = n // P
    iters = (num_chunks + NW - 1) // NW

    mesh = plsc.VectorSubcoreMesh(core_axis_name="c", subcore_axis_name="s")

    scratch = (
        [pltpu.VMEM((3, P), jnp.float32) for _ in range(2)]
        + [pltpu.VMEM((3, P), jnp.float32) for _ in range(2)]
        + [pltpu.VMEM((P, C), jnp.float32) for _ in range(2)]
        + [pltpu.VMEM((P,), jnp.int32) for _ in range(16)]
        + [pltpu.VMEM((P, C), jnp.float32) for _ in range(16)]
        + [pltpu.SemaphoreType.DMA for _ in range(6)]
    )

    @functools.partial(
        pl.kernel,
        out_type=jax.ShapeDtypeStruct((n, C), jnp.float32),
        mesh=mesh,
        compiler_params=pltpu.CompilerParams(use_tc_tiling_on_sc=False),
        scratch_types=scratch,
    )
    def k(xT_hbm, tab_hbm, out_hbm, *s):
        crd = s[0:2]
        wbb = s[2:4]
        outt = s[4:6]
        idx = [s[6:14], s[14:22]]
        rows = [s[22:30], s[30:38]]
        sc_sem = s[38:40]
        sg_sem = s[40:42]
        so_sem = s[42:44]
        wid = lax.axis_index("s") * 2 + lax.axis_index("c")

        def fire_coords(t, b):
            cid = jnp.minimum(wid + t * NW, num_chunks - 1)
            base = pl.multiple_of(cid * P, P)
            pltpu.async_copy(xT_hbm.at[:, pl.ds(base, P)], crd[b], sc_sem[b])

        def jw_maker(b):
            cb, wb = crd[b], wbb[b]
            ib = idx[b]

            def jw_body(j, c2):
                sj = pl.ds(j * L, L)
                fx = (cb[0, sj] + 1.0) * (0.5 * (W - 1))
                fy = (cb[1, sj] + 1.0) * (0.5 * (H - 1))
                fz = (cb[2, sj] + 1.0) * (0.5 * (D - 1))
                fx = jnp.minimum(jnp.maximum(fx, 0.0), float(W - 1))
                fy = jnp.minimum(jnp.maximum(fy, 0.0), float(H - 1))
                fz = jnp.minimum(jnp.maximum(fz, 0.0), float(D - 1))
                x0 = fx.astype(jnp.int32)
                y0 = fy.astype(jnp.int32)
                z0 = fz.astype(jnp.int32)
                wb[0, sj] = fx - x0.astype(jnp.float32)
                wb[1, sj] = fy - y0.astype(jnp.float32)
                wb[2, sj] = fz - z0.astype(jnp.float32)
                x1 = jnp.minimum(x0 + 1, W - 1)
                y1 = jnp.minimum(y0 + 1, H - 1)
                z1 = jnp.minimum(z0 + 1, D - 1)
                zl = z0 * (H * W)
                zh = z1 * (H * W)
                yl = y0 * W
                yh = y1 * W
                ib[0][sj] = zl + yl + x0
                ib[1][sj] = zl + yl + x1
                ib[2][sj] = zl + yh + x0
                ib[3][sj] = zl + yh + x1
                ib[4][sj] = zh + yl + x0
                ib[5][sj] = zh + yl + x1
                ib[6][sj] = zh + yh + x0
                ib[7][sj] = zh + yh + x1
                return c2

            return jw_body

        def comb_maker(b):
            wb = wbb[b]
            rb = rows[b]
            ob = outt[b]

            def comb_body(j, c2):
                sj = pl.ds(j * L, L)
                wxv = wb[0, sj]
                wyv = wb[1, sj]
                wzv = wb[2, sj]
                for q in range(L):
                    p = j * L + q
                    bwx = jnp.full((L,), wxv[q])
                    bwy = jnp.full((L,), wyv[q])
                    bwz = jnp.full((L,), wzv[q])
                    for h in range(C // L):
                        sh = pl.ds(h * L, L)
                        c000 = rb[0][p, sh]
                        c001 = rb[1][p, sh]
                        c010 = rb[2][p, sh]
                        c011 = rb[3][p, sh]
                        c100 = rb[4][p, sh]
                        c101 = rb[5][p, sh]
                        c110 = rb[6][p, sh]
                        c111 = rb[7][p, sh]
                        c00 = c000 + bwx * (c001 - c000)
                        c01 = c010 + bwx * (c011 - c010)
                        c10 = c100 + bwx * (c101 - c100)
                        c11 = c110 + bwx * (c111 - c110)
                        c0 = c00 + bwy * (c01 - c00)
                        c1 = c10 + bwy * (c11 - c10)
                        ob[p, sh] = c0 + bwz * (c1 - c0)
                return c2

            return comb_body

        def stage_a(t, b):
            cid = wid + t * NW

            @pl.when(cid < num_chunks)
            def _():
                pltpu.make_async_copy(
                    xT_hbm.at[:, pl.ds(0, P)], crd[b], sc_sem[b]).wait()
                lax.fori_loop(0, P // L, jw_maker(b), 0)
                for kc in range(8):
                    pltpu.async_copy(
                        tab_hbm.at[idx[b][kc]], rows[b][kc], sg_sem[b])
                fire_coords(t + 2, b)

        def stage_b(t, b):
            cid = wid + t * NW

            @pl.when(cid < num_chunks)
            def _():
                base = pl.multiple_of(cid * P, P)
                for kc in range(8):
                    pltpu.make_async_copy(
                        tab_hbm.at[idx[b][kc]], rows[b][kc], sg_sem[b]).wait()

                @pl.when(t >= 2)
                def _w():
                    pltpu.make_async_copy(
                        out_hbm.at[pl.ds(0, P), :], outt[b], so_sem[b]).wait()

                lax.fori_loop(0, P // L, comb_maker(b), 0)
                pltpu.async_copy(
                    outt[b], out_hbm.at[pl.ds(base, P), :], so_sem[b])

        fire_coords(0, 0)
        fire_coords(1, 1)
        stage_a(0, 0)

        def u_body(u, carry):
            t0 = u * 2
            stage_a(t0 + 1, 1)
            stage_b(t0, 0)
            stage_a(t0 + 2, 0)
            stage_b(t0 + 1, 1)
            return carry

        lax.fori_loop(0, (iters + 1) // 2, u_body, 0)

        for b in (0, 1):
            pltpu.make_async_copy(
                out_hbm.at[pl.ds(0, P), :], outt[b], so_sem[b]).wait()
            pltpu.make_async_copy(
                xT_hbm.at[:, pl.ds(0, P)], crd[b], sc_sem[b]).wait()

    return k(xT, table)


def kernel(x, feature_grid):
    n = x.shape[0]
    n_pad = ((n + P - 1) // P) * P
    table = feature_grid[0].reshape(C, DHW).T
    xT = x.T
    if n_pad != n:
        xT = jnp.pad(xT, ((0, 0), (0, n_pad - n)))
    out = _sc_grid_sample(xT, table, n_pad)
    return out[:n].T.reshape(1, C, 1, 1, n)


# speedup vs baseline: 5.3244x; 1.0005x over previous
import functools

import jax
import jax.numpy as jnp
from jax import lax
from jax.experimental import pallas as pl
from jax.experimental.pallas import tpu as pltpu
from jax.experimental.pallas import tpu_sc as plsc

C = 32
D = 128
H = 128
W = 128
DHW = D * H * W

P = 128
L = 16
NW = 32


def _sc_grid_sample(xT, table, n):
    num_chunks = n // P
    iters = (num_chunks + NW - 1) // NW

    mesh = plsc.VectorSubcoreMesh(core_axis_name="c", subcore_axis_name="s")

    scratch = (
        [pltpu.VMEM((3, P), jnp.float32) for _ in range(2)]
        + [pltpu.VMEM((3, P), jnp.float32) for _ in range(2)]
        + [pltpu.VMEM((P, C), jnp.float32) for _ in range(2)]
        + [pltpu.VMEM((P,), jnp.int32) for _ in range(16)]
        + [pltpu.VMEM((P, C), jnp.float32) for _ in range(16)]
        + [pltpu.SemaphoreType.DMA for _ in range(6)]
    )

    @functools.partial(
        pl.kernel,
        out_type=jax.ShapeDtypeStruct((n, C), jnp.float32),
        mesh=mesh,
        compiler_params=pltpu.CompilerParams(use_tc_tiling_on_sc=False),
        scratch_types=scratch,
    )
    def k(xT_hbm, tab_hbm, out_hbm, *s):
        crd = s[0:2]
        wbb = s[2:4]
        outt = s[4:6]
        idx = [s[6:14], s[14:22]]
        rows = [s[22:30], s[30:38]]
        sc_sem = s[38:40]
        sg_sem = s[40:42]
        so_sem = s[42:44]
        wid = lax.axis_index("s") * 2 + lax.axis_index("c")

        def fire_coords(t, b):
            cid = jnp.minimum(wid + t * NW, num_chunks - 1)
            base = pl.multiple_of(cid * P, P)
            pltpu.async_copy(xT_hbm.at[:, pl.ds(base, P)], crd[b], sc_sem[b])

        def jw_maker(b):
            cb, wb = crd[b], wbb[b]
            ib = idx[b]

            def jw_body(j, c2):
                sj = pl.ds(j * L, L)
                fx = (cb[0, sj] + 1.0) * (0.5 * (W - 1))
                fy = (cb[1, sj] + 1.0) * (0.5 * (H - 1))
                fz = (cb[2, sj] + 1.0) * (0.5 * (D - 1))
                fx = jnp.minimum(jnp.maximum(fx, 0.0), float(W - 1))
                fy = jnp.minimum(jnp.maximum(fy, 0.0), float(H - 1))
                fz = jnp.minimum(jnp.maximum(fz, 0.0), float(D - 1))
                x0 = fx.astype(jnp.int32)
                y0 = fy.astype(jnp.int32)
                z0 = fz.astype(jnp.int32)
                wb[0, sj] = fx - x0.astype(jnp.float32)
                wb[1, sj] = fy - y0.astype(jnp.float32)
                wb[2, sj] = fz - z0.astype(jnp.float32)
                x1 = jnp.minimum(x0 + 1, W - 1)
                y1 = jnp.minimum(y0 + 1, H - 1)
                z1 = jnp.minimum(z0 + 1, D - 1)
                zl = z0 * (H * W)
                zh = z1 * (H * W)
                yl = y0 * W
                yh = y1 * W
                ib[0][sj] = zl + yl + x0
                ib[1][sj] = zl + yl + x1
                ib[2][sj] = zl + yh + x0
                ib[3][sj] = zl + yh + x1
                ib[4][sj] = zh + yl + x0
                ib[5][sj] = zh + yl + x1
                ib[6][sj] = zh + yh + x0
                ib[7][sj] = zh + yh + x1
                return c2

            return jw_body

        def comb_maker(b):
            wb = wbb[b]
            rb = rows[b]
            ob = outt[b]

            def comb_body(j, c2):
                sj = pl.ds(j * L, L)
                wxv = wb[0, sj]
                wyv = wb[1, sj]
                wzv = wb[2, sj]
                for q in range(L):
                    p = j * L + q
                    bwx = jnp.full((L,), wxv[q])
                    bwy = jnp.full((L,), wyv[q])
                    bwz = jnp.full((L,), wzv[q])
                    for h in range(C // L):
                        sh = pl.ds(h * L, L)
                        c000 = rb[0][p, sh]
                        c001 = rb[1][p, sh]
                        c010 = rb[2][p, sh]
                        c011 = rb[3][p, sh]
                        c100 = rb[4][p, sh]
                        c101 = rb[5][p, sh]
                        c110 = rb[6][p, sh]
                        c111 = rb[7][p, sh]
                        c00 = c000 + bwx * (c001 - c000)
                        c01 = c010 + bwx * (c011 - c010)
                        c10 = c100 + bwx * (c101 - c100)
                        c11 = c110 + bwx * (c111 - c110)
                        c0 = c00 + bwy * (c01 - c00)
                        c1 = c10 + bwy * (c11 - c10)
                        ob[p, sh] = c0 + bwz * (c1 - c0)
                return c2

            return comb_body

        def stage_a(t, b):
            cid = wid + t * NW

            @pl.when(cid < num_chunks)
            def _():
                pltpu.make_async_copy(
                    xT_hbm.at[:, pl.ds(0, P)], crd[b], sc_sem[b]).wait()
                lax.fori_loop(0, P // L, jw_maker(b), 0)
                for kc in range(8):
                    pltpu.async_copy(
                        tab_hbm.at[idx[b][kc]], rows[b][kc], sg_sem[b])
                fire_coords(t + 2, b)

        def stage_b(t, b):
            cid = wid + t * NW

            @pl.when(cid < num_chunks)
            def _():
                base = pl.multiple_of(cid * P, P)
                for kc in range(8):
                    pltpu.make_async_copy(
                        tab_hbm.at[idx[b][kc]], rows[b][kc], sg_sem[b]).wait()

                @pl.when(t >= 2)
                def _w():
                    pltpu.make_async_copy(
                        out_hbm.at[pl.ds(0, P), :], outt[b], so_sem[b]).wait()

                lax.fori_loop(0, P // L, comb_maker(b), 0)
                pltpu.async_copy(
                    outt[b], out_hbm.at[pl.ds(base, P), :], so_sem[b])

        fire_coords(0, 0)
        fire_coords(1, 1)
        stage_a(0, 0)

        def u_body(u, carry):
            t0 = u * 2
            stage_a(t0 + 1, 1)
            stage_b(t0, 0)
            stage_a(t0 + 2, 0)
            stage_b(t0 + 1, 1)
            return carry

        lax.fori_loop(0, (iters + 1) // 2, u_body, 0)

        for b in (0, 1):
            pltpu.make_async_copy(
                out_hbm.at[pl.ds(0, P), :], outt[b], so_sem[b]).wait()
            pltpu.make_async_copy(
                xT_hbm.at[:, pl.ds(0, P)], crd[b], sc_sem[b]).wait()

    return k(xT, table)


def kernel(x, feature_grid):
    n = x.shape[0]
    n_pad = ((n + P - 1) // P) * P
    table = jnp.transpose(
        feature_grid, (0, 2, 3, 4, 1)).reshape(DHW, C)
    xT = x.T
    if n_pad != n:
        xT = jnp.pad(xT, ((0, 0), (0, n_pad - n)))
    out = _sc_grid_sample(xT, table, n_pad)
    return out[:n].T.reshape(1, C, 1, 1, n)
